# Initial kernel scaffold; baseline (speedup 1.0000x reference)
import jax
import jax.numpy as jnp
from jax.experimental import pallas as pl


def kernel(inputs):
    raise NotImplementedError("write your pallas kernel here")


# speedup vs baseline: 54.0860x; 54.0860x over previous
import jax
import jax.numpy as jnp
from jax import lax
from jax.experimental import pallas as pl
from jax.experimental.pallas import tpu as pltpu, tpu_sc as plsc

B, S, C = 32, 8192, 128
K = 4
CS = 512
NCHUNK = S // CS
L = 16
NG = C // L


def _kmax_body(x_hbm, out_hbm, buf, obuf):
    cid = lax.axis_index("c")
    sid = lax.axis_index("s")
    wid = sid * 2 + cid

    neg = jnp.full((L,), -jnp.inf, dtype=jnp.float32)

    def row_body(r, st):
        st = list(st)
        for g in range(NG):
            v = buf[r, pl.ds(g * L, L)]
            m1, m2, m3, m4 = st[4 * g:4 * g + 4]
            n1 = jnp.maximum(m1, v)
            t = jnp.minimum(m1, v)
            n2 = jnp.maximum(m2, t)
            t = jnp.minimum(m2, t)
            n3 = jnp.maximum(m3, t)
            t = jnp.minimum(m3, t)
            n4 = jnp.maximum(m4, t)
            st[4 * g:4 * g + 4] = [n1, n2, n3, n4]
        return tuple(st)

    def chunk_body(ci, st):
        pltpu.sync_copy(x_hbm.at[wid, pl.ds(ci * CS, CS)], buf)
        return lax.fori_loop(0, CS, row_body, st)

    init = tuple(neg for _ in range(NG * K))
    st = lax.fori_loop(0, NCHUNK, chunk_body, init)

    for g in range(NG):
        for j in range(K):
            obuf[j, pl.ds(g * L, L)] = st[4 * g + j]
    pltpu.sync_copy(obuf, out_hbm.at[wid])


def kernel(inputs):
    mesh = plsc.VectorSubcoreMesh(core_axis_name="c", subcore_axis_name="s")
    kfn = pl.kernel(
        _kmax_body,
        out_type=jax.ShapeDtypeStruct((B, K, C), jnp.float32),
        mesh=mesh,
        scratch_types=[
            pltpu.VMEM((CS, C), jnp.float32),
            pltpu.VMEM((K, C), jnp.float32),
        ],
    )
    out_kc = kfn(inputs)
    return jnp.transpose(out_kc, (0, 2, 1)).reshape(B, C * K)


# speedup vs baseline: 78.8417x; 1.4577x over previous
import jax
import jax.numpy as jnp
from jax import lax
from jax.experimental import pallas as pl
from jax.experimental.pallas import tpu as pltpu, tpu_sc as plsc

B, S, C = 32, 8192, 128
K = 4
CS = 256
NPAIR = S // (2 * CS)
L = 16
NG = C // L


def _insert_rows(buf, st):
    def row_body(r, st):
        st = list(st)
        for g in range(NG):
            v = buf[r, pl.ds(g * L, L)]
            m1, m2, m3, m4 = st[4 * g:4 * g + 4]
            n1 = jnp.maximum(m1, v)
            t = jnp.minimum(m1, v)
            n2 = jnp.maximum(m2, t)
            t = jnp.minimum(m2, t)
            n3 = jnp.maximum(m3, t)
            t = jnp.minimum(m3, t)
            n4 = jnp.maximum(m4, t)
            st[4 * g:4 * g + 4] = [n1, n2, n3, n4]
        return tuple(st)

    return lax.fori_loop(0, CS, row_body, st)


def _kmax_body(x_hbm, out_hbm, buf0, buf1, obuf, sem0, sem1):
    cid = lax.axis_index("c")
    sid = lax.axis_index("s")
    wid = sid * 2 + cid

    pltpu.async_copy(x_hbm.at[wid, pl.ds(0, CS)], buf0, sem0)
    pltpu.async_copy(x_hbm.at[wid, pl.ds(CS, CS)], buf1, sem1)

    neg = jnp.full((L,), -jnp.inf, dtype=jnp.float32)

    def pair_body(i, st):
        base = 2 * i * CS
        pltpu.make_async_copy(x_hbm.at[wid, pl.ds(0, CS)], buf0, sem0).wait()
        st = _insert_rows(buf0, st)

        @pl.when(i < NPAIR - 1)
        def _():
            pltpu.async_copy(
                x_hbm.at[wid, pl.ds(base + 2 * CS, CS)], buf0, sem0)

        pltpu.make_async_copy(x_hbm.at[wid, pl.ds(0, CS)], buf1, sem1).wait()
        st = _insert_rows(buf1, st)

        @pl.when(i < NPAIR - 1)
        def _():
            pltpu.async_copy(
                x_hbm.at[wid, pl.ds(base + 3 * CS, CS)], buf1, sem1)

        return st

    init = tuple(neg for _ in range(NG * K))
    st = lax.fori_loop(0, NPAIR, pair_body, init)

    for g in range(NG):
        for j in range(K):
            obuf[j, pl.ds(g * L, L)] = st[4 * g + j]
    pltpu.sync_copy(obuf, out_hbm.at[wid])


def kernel(inputs):
    mesh = plsc.VectorSubcoreMesh(core_axis_name="c", subcore_axis_name="s")
    kfn = pl.kernel(
        _kmax_body,
        out_type=jax.ShapeDtypeStruct((B, K, C), jnp.float32),
        mesh=mesh,
        scratch_types=[
            pltpu.VMEM((CS, C), jnp.float32),
            pltpu.VMEM((CS, C), jnp.float32),
            pltpu.VMEM((K, C), jnp.float32),
            pltpu.SemaphoreType.DMA,
            pltpu.SemaphoreType.DMA,
        ],
    )
    out_kc = kfn(inputs)
    return jnp.transpose(out_kc, (0, 2, 1)).reshape(B, C * K)


# speedup vs baseline: 81.6938x; 1.0362x over previous
import jax
import jax.numpy as jnp
from jax import lax
from jax.experimental import pallas as pl
from jax.experimental.pallas import tpu as pltpu, tpu_sc as plsc

B, S, C = 32, 8192, 128
K = 4
CS = 256
NPAIR = S // (2 * CS)
L = 16
NG = C // L


def _insert_rows(buf, st):
    def row_body(r, st):
        st = list(st)
        for g in range(NG):
            v0 = buf[4 * r, pl.ds(g * L, L)]
            v1 = buf[4 * r + 1, pl.ds(g * L, L)]
            v2 = buf[4 * r + 2, pl.ds(g * L, L)]
            v3 = buf[4 * r + 3, pl.ds(g * L, L)]
            a = jnp.minimum(v0, v1)
            b = jnp.maximum(v0, v1)
            c = jnp.minimum(v2, v3)
            d = jnp.maximum(v2, v3)
            e0 = jnp.minimum(a, c)
            t1 = jnp.maximum(a, c)
            e3 = jnp.maximum(b, d)
            t2 = jnp.minimum(b, d)
            e1 = jnp.minimum(t1, t2)
            e2 = jnp.maximum(t1, t2)
            s1, s2, s3, s4 = st[4 * g:4 * g + 4]
            b0 = jnp.maximum(s1, e0)
            b1 = jnp.maximum(s2, e1)
            b2 = jnp.maximum(s3, e2)
            b3 = jnp.maximum(s4, e3)
            u0 = jnp.maximum(b0, b2)
            u2 = jnp.minimum(b0, b2)
            u1 = jnp.maximum(b1, b3)
            u3 = jnp.minimum(b1, b3)
            st[4 * g:4 * g + 4] = [
                jnp.maximum(u0, u1), jnp.minimum(u0, u1),
                jnp.maximum(u2, u3), jnp.minimum(u2, u3)]
        return tuple(st)

    return lax.fori_loop(0, CS // 4, row_body, st)


def _kmax_body(x_hbm, out_hbm, buf0, buf1, obuf, sem0, sem1):
    cid = lax.axis_index("c")
    sid = lax.axis_index("s")
    wid = sid * 2 + cid

    pltpu.async_copy(x_hbm.at[wid, pl.ds(0, CS)], buf0, sem0)
    pltpu.async_copy(x_hbm.at[wid, pl.ds(CS, CS)], buf1, sem1)

    neg = jnp.full((L,), -jnp.inf, dtype=jnp.float32)

    def pair_body(i, st):
        base = 2 * i * CS
        pltpu.make_async_copy(x_hbm.at[wid, pl.ds(0, CS)], buf0, sem0).wait()
        st = _insert_rows(buf0, st)

        @pl.when(i < NPAIR - 1)
        def _():
            pltpu.async_copy(
                x_hbm.at[wid, pl.ds(base + 2 * CS, CS)], buf0, sem0)

        pltpu.make_async_copy(x_hbm.at[wid, pl.ds(0, CS)], buf1, sem1).wait()
        st = _insert_rows(buf1, st)

        @pl.when(i < NPAIR - 1)
        def _():
            pltpu.async_copy(
                x_hbm.at[wid, pl.ds(base + 3 * CS, CS)], buf1, sem1)

        return st

    init = tuple(neg for _ in range(NG * K))
    st = lax.fori_loop(0, NPAIR, pair_body, init)

    for g in range(NG):
        for j in range(K):
            obuf[j, pl.ds(g * L, L)] = st[4 * g + j]
    pltpu.sync_copy(obuf, out_hbm.at[wid])


def kernel(inputs):
    mesh = plsc.VectorSubcoreMesh(core_axis_name="c", subcore_axis_name="s")
    kfn = pl.kernel(
        _kmax_body,
        out_type=jax.ShapeDtypeStruct((B, K, C), jnp.float32),
        mesh=mesh,
        scratch_types=[
            pltpu.VMEM((CS, C), jnp.float32),
            pltpu.VMEM((CS, C), jnp.float32),
            pltpu.VMEM((K, C), jnp.float32),
            pltpu.SemaphoreType.DMA,
            pltpu.SemaphoreType.DMA,
        ],
    )
    out_kc = kfn(inputs)
    return jnp.transpose(out_kc, (0, 2, 1)).reshape(B, C * K)
